# baseline (speedup 1.0000x reference)
import math

import jax
import jax.numpy as jnp
from jax.experimental import pallas as pl
from jax.experimental.pallas import tpu as pltpu

_PAD = 0
_CONF = 0.9
_N = 2048
_V = 32000
_W = 3200
_GRID = _V // _W
_SLABS = _W // 128

_L1 = _CONF
_L0 = (1.0 - _CONF) / (_V - 2)
_C = _L1 * math.log(_L1) + (_V - 1) * _L0 * math.log(_L0)


def _body(yts_ref, m_ref, yp_ref, loss_ref, npad_ref, d_ref, s_ref, g_ref):
    j = pl.program_id(0)

    @pl.when(j == 0)
    def _():
        lane = jax.lax.broadcasted_iota(jnp.int32, (_N, 128), 1)
        d_ref[...] = yts_ref[...] - lane
        s_ref[...] = jnp.zeros((_N, 128), jnp.float32)
        g_ref[...] = jnp.zeros((_N, 128), jnp.float32)

    d = d_ref[...]
    part_s = yp_ref[:, 0:128]
    part_g = jnp.where(d == j * _W, part_s, 0.0)
    for c in range(1, _SLABS):
        slab = yp_ref[:, c * 128:(c + 1) * 128]
        part_s = part_s + slab
        part_g = part_g + jnp.where(d == j * _W + c * 128, slab, 0.0)
    s_ref[...] += part_s
    g_ref[...] += part_g

    @pl.when(j == _GRID - 1)
    def _():
        m = m_ref[...]
        npad = jnp.sum(m)
        total_s = jnp.sum(s_ref[...] * m)
        total_g = jnp.sum(g_ref[...])
        npad_ref[0, 0] = npad.astype(jnp.int32)
        loss_ref[0, 0] = npad * _C - _L0 * total_s - (_L1 - _L0) * total_g


def kernel(y_pred, y_true):
    yp = y_pred.reshape(_N, _V)
    yt = y_true.reshape(_N, 1)
    nonpad = yt != _PAD
    yts = jnp.where(nonpad, yt, -1)
    mrow = nonpad.astype(jnp.float32)

    loss, npad = pl.pallas_call(
        _body,
        grid=(_GRID,),
        in_specs=[
            pl.BlockSpec((_N, 1), lambda j: (0, 0)),
            pl.BlockSpec((_N, 1), lambda j: (0, 0)),
            pl.BlockSpec((_N, _W), lambda j: (0, j)),
        ],
        out_specs=[
            pl.BlockSpec(memory_space=pltpu.SMEM),
            pl.BlockSpec(memory_space=pltpu.SMEM),
        ],
        out_shape=[
            jax.ShapeDtypeStruct((1, 1), jnp.float32),
            jax.ShapeDtypeStruct((1, 1), jnp.int32),
        ],
        scratch_shapes=[
            pltpu.VMEM((_N, 128), jnp.int32),
            pltpu.VMEM((_N, 128), jnp.float32),
            pltpu.VMEM((_N, 128), jnp.float32),
        ],
    )(yts, mrow, yp)
    return (loss[0, 0], npad[0, 0])
